# Initial kernel scaffold; baseline (speedup 1.0000x reference)
import jax
import jax.numpy as jnp
from jax.experimental import pallas as pl


def kernel(x, edge_index, edge_weight, Ws, bs, bn_g, bn_b, fin_g, fin_b, mW1, mb1, mg, mbt, mW2, mb2):
    raise NotImplementedError("write your pallas kernel here")


# speedup vs baseline: 4.2494x; 4.2494x over previous
import functools

import jax
import jax.numpy as jnp
from jax import lax
from jax.experimental import pallas as pl
from jax.experimental.pallas import tpu as pltpu
from jax.experimental.pallas import tpu_sc as plsc

N = 50000
E = 800000
NP = 51200
EP = 819200
HID = 64
OUT = 16
MLP_HID = 256
NREAL = float(N)
EPS = 1e-5

F32 = jnp.float32
I32 = jnp.int32


def _dense(parts, W, g, beta, bias, relu_in, out_widths):
    n_parts = len(parts)
    pw = [int(p.shape[1]) for p in parts]
    FW = sum(pw)
    DBR = 2048
    NB = NP // DBR

    def body(*refs):
        hrefs = refs[:n_parts]
        Wr, gr, br, bir = refs[n_parts:n_parts + 4]
        outs = refs[n_parts + 4:-1]
        st = refs[-1]
        ph = pl.program_id(0)
        i = pl.program_id(1)
        if n_parts > 1:
            a = jnp.concatenate([r[...] for r in hrefs], axis=1)
        else:
            a = hrefs[0][...]
        if relu_in:
            a = jnp.maximum(a, 0.0)

        @pl.when(ph == 0)
        def _():
            @pl.when(i == 0)
            def _():
                st[...] = jnp.zeros_like(st)
            st[0:1, :] += jnp.sum(a, axis=0, keepdims=True)
            st[1:2, :] += jnp.sum(a * a, axis=0, keepdims=True)

        @pl.when(ph == 1)
        def _():
            mu = st[0:1, :] / NREAL
            var = st[1:2, :] / NREAL - mu * mu
            sc = gr[...] * lax.rsqrt(var + EPS)
            t = br[...] - mu * sc
            y = jnp.dot(a * sc + t, Wr[...], preferred_element_type=F32)
            y = y + bir[...]
            off = 0
            for oref, ow in zip(outs, out_widths):
                oref[...] = y[:, off:off + ow]
                off += ow

    def _rowspec(w_):
        return pl.BlockSpec((DBR, w_), lambda p, i: (i, 0))

    def _full(shape):
        return pl.BlockSpec(shape, lambda p, i: (0, 0))

    in_specs = [_rowspec(w_) for w_ in pw] + [
        _full(W.shape), _full(g.shape), _full(beta.shape), _full(bias.shape)]
    out_specs = [_rowspec(ow) for ow in out_widths]
    out_shape = [jax.ShapeDtypeStruct((NP, ow), F32) for ow in out_widths]
    return pl.pallas_call(
        body,
        grid=(2, NB),
        in_specs=in_specs,
        out_specs=out_specs,
        out_shape=out_shape,
        scratch_shapes=[pltpu.VMEM((2, FW), F32)],
    )(*parts, W, g, beta, bias)


def _hf_apply(p0, p1, g, beta):
    DBR = 2048
    NB = NP // DBR

    def body(a_ref, b_ref, gr, br, out_ref, st):
        ph = pl.program_id(0)
        i = pl.program_id(1)
        a = jnp.maximum(a_ref[...] + b_ref[...], 0.0)

        @pl.when(ph == 0)
        def _():
            @pl.when(i == 0)
            def _():
                st[...] = jnp.zeros_like(st)
            st[0:1, :] += jnp.sum(a, axis=0, keepdims=True)
            st[1:2, :] += jnp.sum(a * a, axis=0, keepdims=True)

        @pl.when(ph == 1)
        def _():
            mu = st[0:1, :] / NREAL
            var = st[1:2, :] / NREAL - mu * mu
            sc = gr[...] * lax.rsqrt(var + EPS)
            t = br[...] - mu * sc
            out_ref[...] = a * sc + t

    rowspec = pl.BlockSpec((DBR, OUT), lambda p, i: (i, 0))
    full = pl.BlockSpec((1, OUT), lambda p, i: (0, 0))
    return pl.pallas_call(
        body,
        grid=(2, NB),
        in_specs=[rowspec, rowspec, full, full],
        out_specs=rowspec,
        out_shape=jax.ShapeDtypeStruct((NP, OUT), F32),
        scratch_shapes=[pltpu.VMEM((2, OUT), F32)],
    )(p0, p1, g, beta)


def _make_sc64():
    mesh = plsc.VectorSubcoreMesh(core_axis_name="c", subcore_axis_name="s")
    T = EP // 16
    OCH = 2048
    NO = T // OCH
    NJ = OCH // 128
    RPT = NP // 16

    @functools.partial(
        pl.kernel, mesh=mesh,
        compiler_params=pltpu.CompilerParams(use_tc_tiling_on_sc=False),
        out_type=(jax.ShapeDtypeStruct((NP, 32), F32),
                  jax.ShapeDtypeStruct((NP, 32), F32)),
        scratch_types=[
            pltpu.VMEM((OCH,), I32),
            pltpu.VMEM((NJ, 128), I32),
            pltpu.VMEM((OCH,), F32),
            pltpu.VMEM((128, 32), F32),
            pltpu.VMEM((128, 32), F32),
            pltpu.VMEM_SHARED((NP, 32), F32),
            pltpu.SemaphoreType.DMA,
        ])
    def sck(hlo, hhi, rowp, col2, wp, z128,
            out_lo, out_hi, rowb, colb, wb, gbuf, zbuf, acc, sem):
        c = lax.axis_index("c")
        s = lax.axis_index("s")
        pltpu.sync_copy(z128, zbuf)
        for j in range(RPT // 128):
            pltpu.sync_copy(zbuf, acc.at[pl.ds(s * RPT + j * 128, 128)])
        plsc.subcore_barrier()

        def outer(o, carry):
            eb = s * T + o * OCH
            pltpu.sync_copy(rowp.at[pl.ds(eb, OCH)], rowb)
            pltpu.sync_copy(col2.at[pl.ds(s * (T // 128) + o * NJ, NJ)], colb)
            pltpu.sync_copy(wp.at[pl.ds(eb, OCH)], wb)
            for j in range(NJ):
                idxs = rowb.at[pl.ds(j * 128, 128)]

                @pl.when(c == 0)
                def _():
                    pltpu.async_copy(hlo.at[idxs], gbuf, sem).wait()

                @pl.when(c == 1)
                def _():
                    pltpu.async_copy(hhi.at[idxs], gbuf, sem).wait()

                def scale(g, cc):
                    wvec = wb[pl.ds(j * 128 + g * 16, 16)]
                    for k in range(16):
                        e = g * 16 + k
                        wv = wvec[k]
                        gbuf[e, pl.ds(0, 16)] = gbuf[e, pl.ds(0, 16)] * wv
                        gbuf[e, pl.ds(16, 16)] = gbuf[e, pl.ds(16, 16)] * wv
                    return cc

                lax.fori_loop(0, 8, scale, 0)
                pltpu.sync_copy(gbuf, acc.at[colb.at[j]], add=True)
            return carry

        lax.fori_loop(0, NO, outer, 0)
        plsc.subcore_barrier()

        @pl.when(c == 0)
        def _():
            pltpu.sync_copy(acc.at[pl.ds(s * RPT, RPT)],
                            out_lo.at[pl.ds(s * RPT, RPT)])

        @pl.when(c == 1)
        def _():
            pltpu.sync_copy(acc.at[pl.ds(s * RPT, RPT)],
                            out_hi.at[pl.ds(s * RPT, RPT)])

    return sck


def _make_sc16():
    mesh = plsc.VectorSubcoreMesh(core_axis_name="c", subcore_axis_name="s")
    T = EP // 32
    OCH = 2560
    NO = T // OCH
    NJ = OCH // 128
    RPT = NP // 16

    @functools.partial(
        pl.kernel, mesh=mesh,
        compiler_params=pltpu.CompilerParams(use_tc_tiling_on_sc=False),
        out_type=(jax.ShapeDtypeStruct((NP, OUT), F32),
                  jax.ShapeDtypeStruct((NP, OUT), F32)),
        scratch_types=[
            pltpu.VMEM((OCH,), I32),
            pltpu.VMEM((NJ, 128), I32),
            pltpu.VMEM((OCH,), F32),
            pltpu.VMEM((128, OUT), F32),
            pltpu.VMEM((128, OUT), F32),
            pltpu.VMEM_SHARED((NP, OUT), F32),
            pltpu.SemaphoreType.DMA,
        ])
    def sck(h16, rowp, col2, wp, z16,
            out0, out1, rowb, colb, wb, gbuf, zbuf, acc, sem):
        c = lax.axis_index("c")
        s = lax.axis_index("s")
        w = c * 16 + s
        pltpu.sync_copy(z16, zbuf)
        for j in range(RPT // 128):
            pltpu.sync_copy(zbuf, acc.at[pl.ds(s * RPT + j * 128, 128)])
        plsc.subcore_barrier()

        def outer(o, carry):
            eb = w * T + o * OCH
            pltpu.sync_copy(rowp.at[pl.ds(eb, OCH)], rowb)
            pltpu.sync_copy(col2.at[pl.ds(w * (T // 128) + o * NJ, NJ)], colb)
            pltpu.sync_copy(wp.at[pl.ds(eb, OCH)], wb)
            for j in range(NJ):
                idxs = rowb.at[pl.ds(j * 128, 128)]
                pltpu.async_copy(h16.at[idxs], gbuf, sem).wait()

                def scale(g, cc):
                    wvec = wb[pl.ds(j * 128 + g * 16, 16)]
                    for k in range(16):
                        e = g * 16 + k
                        gbuf[e, pl.ds(0, 16)] = gbuf[e, pl.ds(0, 16)] * wvec[k]
                    return cc

                lax.fori_loop(0, 8, scale, 0)
                pltpu.sync_copy(gbuf, acc.at[colb.at[j]], add=True)
            return carry

        lax.fori_loop(0, NO, outer, 0)
        plsc.subcore_barrier()

        @pl.when(c == 0)
        def _():
            pltpu.sync_copy(acc.at[pl.ds(s * RPT, RPT)],
                            out0.at[pl.ds(s * RPT, RPT)])

        @pl.when(c == 1)
        def _():
            pltpu.sync_copy(acc.at[pl.ds(s * RPT, RPT)],
                            out1.at[pl.ds(s * RPT, RPT)])

    return sck


def _make_scgather():
    mesh = plsc.VectorSubcoreMesh(core_axis_name="c", subcore_axis_name="s")
    T = EP // 32
    OCH = 2560
    NO = T // OCH
    NJ = OCH // 128

    @functools.partial(
        pl.kernel, mesh=mesh,
        compiler_params=pltpu.CompilerParams(use_tc_tiling_on_sc=False),
        out_type=(jax.ShapeDtypeStruct((EP, OUT), F32),
                  jax.ShapeDtypeStruct((EP, OUT), F32)),
        scratch_types=[
            pltpu.VMEM((OCH,), I32),
            pltpu.VMEM((NJ, 128), I32),
            pltpu.VMEM((128, OUT), F32),
            pltpu.SemaphoreType.DMA,
        ])
    def gk(hf, rowp, col2, fr, fc, rowb, colb, buf, sem):
        c = lax.axis_index("c")
        s = lax.axis_index("s")
        w = c * 16 + s

        def outer(o, carry):
            eb = w * T + o * OCH
            pltpu.sync_copy(rowp.at[pl.ds(eb, OCH)], rowb)
            pltpu.sync_copy(col2.at[pl.ds(w * (T // 128) + o * NJ, NJ)], colb)
            for j in range(NJ):
                pltpu.async_copy(hf.at[rowb.at[pl.ds(j * 128, 128)]],
                                 buf, sem).wait()
                pltpu.sync_copy(buf, fr.at[pl.ds(eb + j * 128, 128)])
                pltpu.async_copy(hf.at[colb.at[j]], buf, sem).wait()
                pltpu.sync_copy(buf, fc.at[pl.ds(eb + j * 128, 128)])
            return carry

        lax.fori_loop(0, NO, outer, 0)

    return gk


_sc64 = _make_sc64()
_sc16 = _make_sc16()
_scgather = _make_scgather()


def _edge_stats(fr, fc):
    BR = 2000
    NB = E // BR

    def body(frr, fcr, so, xo, ssum, sxtx):
        i = pl.program_id(0)
        e = jnp.concatenate([frr[...], fcr[...]], axis=1)

        @pl.when(i == 0)
        def _():
            ssum[...] = jnp.zeros_like(ssum)
            sxtx[...] = jnp.zeros_like(sxtx)

        ssum[...] += jnp.sum(e, axis=0, keepdims=True)
        sxtx[...] += lax.dot_general(e, e, (((0,), (0,)), ((), ())),
                                     preferred_element_type=F32)

        @pl.when(i == NB - 1)
        def _():
            so[...] = ssum[...]
            xo[...] = sxtx[...]

    rowspec = pl.BlockSpec((BR, OUT), lambda i: (i, 0))
    return pl.pallas_call(
        body,
        grid=(NB,),
        in_specs=[rowspec, rowspec],
        out_specs=[pl.BlockSpec((1, 32), lambda i: (0, 0)),
                   pl.BlockSpec((32, 32), lambda i: (0, 0))],
        out_shape=[jax.ShapeDtypeStruct((1, 32), F32),
                   jax.ShapeDtypeStruct((32, 32), F32)],
        scratch_shapes=[pltpu.VMEM((1, 32), F32), pltpu.VMEM((32, 32), F32)],
    )(fr, fc)


def _edge_mlp(fr, fc, Ws1, t1, Ws2, t2, mW2, mb2):
    BR = 2048
    NB = EP // BR

    def body(frr, fcr, w1r, t1r, w2r, t2r, mwr, mbr, out_ref):
        e = jnp.concatenate([frr[...], fcr[...]], axis=1)
        a1 = jnp.maximum(jnp.dot(e, w1r[...], preferred_element_type=F32)
                         + t1r[...], 0.0)
        a2 = jnp.maximum(jnp.dot(e, w2r[...], preferred_element_type=F32)
                         + t2r[...], 0.0)
        p = (jnp.dot(a1, mwr[...], preferred_element_type=F32)
             + jnp.dot(a2, mwr[...], preferred_element_type=F32)) * 0.5
        p = p + mbr[...]
        out_ref[...] = (1.0 / (1.0 + jnp.exp(-p))).reshape(BR)

    rowspec = pl.BlockSpec((BR, OUT), lambda i: (i, 0))

    def _full(shape):
        return pl.BlockSpec(shape, lambda i: tuple(0 for _ in shape))

    return pl.pallas_call(
        body,
        grid=(NB,),
        in_specs=[rowspec, rowspec, _full(Ws1.shape), _full(t1.shape),
                  _full(Ws2.shape), _full(t2.shape), _full(mW2.shape),
                  _full(mb2.shape)],
        out_specs=pl.BlockSpec((BR,), lambda i: (i,)),
        out_shape=jax.ShapeDtypeStruct((EP,), F32),
    )(fr, fc, Ws1, t1, Ws2, t2, mW2, mb2)


def kernel(x, edge_index, edge_weight, Ws, bs, bn_g, bn_b, fin_g, fin_b,
           mW1, mb1, mg, mbt, mW2, mb2):
    row = edge_index[0].astype(I32)
    col = edge_index[1].astype(I32)
    npad = EP - E
    it = jnp.arange(npad, dtype=I32)
    rowp = jnp.concatenate([row, it % 8192])
    colp = jnp.concatenate([col, N + (it % 1024)])
    wp = jnp.concatenate([edge_weight, jnp.zeros((npad,), F32)])
    col2 = colp.reshape(EP // 128, 128)
    z128 = jnp.zeros((128, 32), F32)
    z16 = jnp.zeros((128, OUT), F32)

    xp = jnp.zeros((NP, 8), F32).at[:N, :2].set(x)
    W0 = jnp.zeros((8, HID), F32).at[:2].set(Ws[0])
    g0 = jnp.ones((1, 8), F32).at[0, :2].set(bn_g[0])
    bb0 = jnp.zeros((1, 8), F32).at[0, :2].set(bn_b[0])
    hlo, hhi = _dense([xp], W0, g0, bb0, bs[0].reshape(1, HID),
                      relu_in=False, out_widths=(32, 32))
    hlo, hhi = _sc64(hlo, hhi, rowp, col2, wp, z128)

    Wst = jnp.stack(Ws[1:11])
    bst = jnp.stack(bs[1:11]).reshape(10, 1, HID)
    gst = jnp.stack(bn_g[1:11]).reshape(10, 1, HID)
    nst = jnp.stack(bn_b[1:11]).reshape(10, 1, HID)

    def step(carry, ws):
        lo, hi = carry
        W, b, g, bb = ws
        lo2, hi2 = _dense([lo, hi], W, g, bb, b,
                          relu_in=True, out_widths=(32, 32))
        lo3, hi3 = _sc64(lo2, hi2, rowp, col2, wp, z128)
        return (lo3, hi3), None

    (hlo, hhi), _ = lax.scan(step, (hlo, hhi), (Wst, bst, gst, nst))

    h16 = _dense([hlo, hhi], Ws[11], bn_g[11].reshape(1, HID),
                 bn_b[11].reshape(1, HID), bs[11].reshape(1, OUT),
                 relu_in=True, out_widths=(OUT,))[0]
    p0, p1 = _sc16(h16, rowp, col2, wp, z16)
    hf = _hf_apply(p0, p1, fin_g.reshape(1, OUT), fin_b.reshape(1, OUT))

    fr, fc = _scgather(hf, rowp, col2)
    sums, xtx = _edge_stats(fr, fc)
    mu_e = sums.reshape(32) / E
    C = xtx / E - jnp.outer(mu_e, mu_e)
    W1r = jnp.roll(mW1, OUT, axis=0)

    def fold(W1):
        mu_z = mu_e @ W1 + mb1
        var_z = jnp.sum(W1 * (C @ W1), axis=0)
        s = mg / jnp.sqrt(var_z + EPS)
        t = (mb1 - mu_z) * s + mbt
        return W1 * s[None, :], t.reshape(1, MLP_HID)

    Ws1, t1 = fold(mW1)
    Ws2, t2 = fold(W1r)
    out = _edge_mlp(fr, fc, Ws1, t1, Ws2, t2, mW2, mb2.reshape(1, 1))
    return out[:E]


# speedup vs baseline: 6.1460x; 1.4463x over previous
import functools

import jax
import jax.numpy as jnp
from jax import lax
from jax.experimental import pallas as pl
from jax.experimental.pallas import tpu as pltpu
from jax.experimental.pallas import tpu_sc as plsc

N = 50000
E = 800000
NP = 51200
EP = 819200
HID = 64
OUT = 16
MLP_HID = 256
NREAL = float(N)
EPS = 1e-5

F32 = jnp.float32
I32 = jnp.int32


def _dense(parts, W, g, beta, bias, relu_in, out_widths):
    n_parts = len(parts)
    pw = [int(p.shape[1]) for p in parts]
    FW = sum(pw)
    DBR = 2048
    NB = NP // DBR

    def body(*refs):
        hrefs = refs[:n_parts]
        Wr, gr, br, bir = refs[n_parts:n_parts + 4]
        outs = refs[n_parts + 4:-1]
        st = refs[-1]
        ph = pl.program_id(0)
        i = pl.program_id(1)
        if n_parts > 1:
            a = jnp.concatenate([r[...] for r in hrefs], axis=1)
        else:
            a = hrefs[0][...]
        if relu_in:
            a = jnp.maximum(a, 0.0)

        @pl.when(ph == 0)
        def _():
            @pl.when(i == 0)
            def _():
                st[...] = jnp.zeros_like(st)
            st[0:1, :] += jnp.sum(a, axis=0, keepdims=True)
            st[1:2, :] += jnp.sum(a * a, axis=0, keepdims=True)

        @pl.when(ph == 1)
        def _():
            mu = st[0:1, :] / NREAL
            var = st[1:2, :] / NREAL - mu * mu
            sc = gr[...] * lax.rsqrt(var + EPS)
            t = br[...] - mu * sc
            y = jnp.dot(a * sc + t, Wr[...], preferred_element_type=F32)
            y = y + bir[...]
            off = 0
            for oref, ow in zip(outs, out_widths):
                oref[...] = y[:, off:off + ow]
                off += ow

    def _rowspec(w_):
        return pl.BlockSpec((DBR, w_), lambda p, i: (i, 0))

    def _full(shape):
        return pl.BlockSpec(shape, lambda p, i: (0, 0))

    in_specs = [_rowspec(w_) for w_ in pw] + [
        _full(W.shape), _full(g.shape), _full(beta.shape), _full(bias.shape)]
    out_specs = [_rowspec(ow) for ow in out_widths]
    out_shape = [jax.ShapeDtypeStruct((NP, ow), F32) for ow in out_widths]
    return pl.pallas_call(
        body,
        grid=(2, NB),
        in_specs=in_specs,
        out_specs=out_specs,
        out_shape=out_shape,
        scratch_shapes=[pltpu.VMEM((2, FW), F32)],
    )(*parts, W, g, beta, bias)


def _hf_apply(p0, p1, g, beta):
    DBR = 2048
    NB = NP // DBR

    def body(a_ref, b_ref, gr, br, out_ref, st):
        ph = pl.program_id(0)
        i = pl.program_id(1)
        a = jnp.maximum(a_ref[...] + b_ref[...], 0.0)[:, :OUT]

        @pl.when(ph == 0)
        def _():
            @pl.when(i == 0)
            def _():
                st[...] = jnp.zeros_like(st)
            st[0:1, :] += jnp.sum(a, axis=0, keepdims=True)
            st[1:2, :] += jnp.sum(a * a, axis=0, keepdims=True)

        @pl.when(ph == 1)
        def _():
            mu = st[0:1, :] / NREAL
            var = st[1:2, :] / NREAL - mu * mu
            sc = gr[...] * lax.rsqrt(var + EPS)
            t = br[...] - mu * sc
            out_ref[...] = a * sc + t

    inspec = pl.BlockSpec((DBR, 32), lambda p, i: (i, 0))
    rowspec = pl.BlockSpec((DBR, OUT), lambda p, i: (i, 0))
    full = pl.BlockSpec((1, OUT), lambda p, i: (0, 0))
    return pl.pallas_call(
        body,
        grid=(2, NB),
        in_specs=[inspec, inspec, full, full],
        out_specs=rowspec,
        out_shape=jax.ShapeDtypeStruct((NP, OUT), F32),
        scratch_shapes=[pltpu.VMEM((2, OUT), F32)],
    )(p0, p1, g, beta)


def _make_sc64():
    mesh = plsc.VectorSubcoreMesh(core_axis_name="c", subcore_axis_name="s")
    T = EP // 16
    BIG = 2560
    C = 256
    NK = BIG // C
    NO = T // BIG
    RPT = NP // 16

    @functools.partial(
        pl.kernel, mesh=mesh,
        compiler_params=pltpu.CompilerParams(use_tc_tiling_on_sc=False),
        out_type=(jax.ShapeDtypeStruct((NP, 32), F32),
                  jax.ShapeDtypeStruct((NP, 32), F32)),
        scratch_types=[
            pltpu.VMEM((BIG,), I32),
            pltpu.VMEM((NK, C), I32),
            pltpu.VMEM((BIG,), F32),
            pltpu.VMEM((C, 32), F32),
            pltpu.VMEM((C, 32), F32),
            pltpu.VMEM((128, 32), F32),
            pltpu.VMEM_SHARED((NP, 32), F32),
            pltpu.SemaphoreType.DMA,
            pltpu.SemaphoreType.DMA,
        ])
    def sck(hlo, hhi, rowp, col2, wp, z128,
            out_lo, out_hi, rowb, colb, wb, gbufa, gbufb, zbuf, acc,
            gsema, gsemb):
        c = lax.axis_index("c")
        s = lax.axis_index("s")
        pltpu.sync_copy(z128, zbuf)
        for j in range(RPT // 128):
            pltpu.sync_copy(zbuf, acc.at[pl.ds(s * RPT + j * 128, 128)])
        plsc.subcore_barrier()

        gbufs = (gbufa, gbufb)
        gsems = (gsema, gsemb)

        def gissue(k, gbuf, gsem):
            @pl.when(c == 0)
            def _():
                pltpu.async_copy(hlo.at[rowb.at[pl.ds(k * C, C)]], gbuf, gsem)

            @pl.when(c == 1)
            def _():
                pltpu.async_copy(hhi.at[rowb.at[pl.ds(k * C, C)]], gbuf, gsem)

        def outer(o, carry):
            eb = s * T + o * BIG
            pltpu.sync_copy(rowp.at[pl.ds(eb, BIG)], rowb)
            pltpu.sync_copy(col2.at[pl.ds(eb // C, NK)], colb)
            pltpu.sync_copy(wp.at[pl.ds(eb, BIG)], wb)
            gissue(0, gbufs[0], gsems[0])
            for k in range(NK):
                gbuf, gsem = gbufs[k % 2], gsems[k % 2]
                if k + 1 < NK:
                    gissue(k + 1, gbufs[(k + 1) % 2], gsems[(k + 1) % 2])
                pltpu.make_async_copy(
                    hlo.at[rowb.at[pl.ds(k * C, C)]], gbuf, gsem).wait()

                def scale(g, cc):
                    wvec = wb[pl.ds(k * C + g * 16, 16)]
                    for kk in range(16):
                        e = g * 16 + kk
                        wv = wvec[kk]
                        gbuf[e, pl.ds(0, 16)] = gbuf[e, pl.ds(0, 16)] * wv
                        gbuf[e, pl.ds(16, 16)] = gbuf[e, pl.ds(16, 16)] * wv
                    return cc

                lax.fori_loop(0, C // 16, scale, 0)
                pltpu.sync_copy(gbuf, acc.at[colb.at[k]], add=True)
            return carry

        lax.fori_loop(0, NO, outer, 0)
        plsc.subcore_barrier()

        @pl.when(c == 0)
        def _():
            pltpu.sync_copy(acc.at[pl.ds(s * RPT, RPT)],
                            out_lo.at[pl.ds(s * RPT, RPT)])

        @pl.when(c == 1)
        def _():
            pltpu.sync_copy(acc.at[pl.ds(s * RPT, RPT)],
                            out_hi.at[pl.ds(s * RPT, RPT)])

    return sck


def _make_scgather():
    mesh = plsc.VectorSubcoreMesh(core_axis_name="c", subcore_axis_name="s")
    T = EP // 32
    OCH = 2560
    C = 1280
    NO = T // OCH

    @functools.partial(
        pl.kernel, mesh=mesh,
        compiler_params=pltpu.CompilerParams(use_tc_tiling_on_sc=False),
        out_type=(jax.ShapeDtypeStruct((EP, OUT), F32),
                  jax.ShapeDtypeStruct((EP, OUT), F32)),
        scratch_types=[
            pltpu.VMEM((OCH,), I32),
            pltpu.VMEM((OCH,), I32),
            pltpu.VMEM((C, OUT), F32),
            pltpu.VMEM((C, OUT), F32),
            pltpu.VMEM((C, OUT), F32),
            pltpu.VMEM((C, OUT), F32),
            pltpu.SemaphoreType.DMA,
            pltpu.SemaphoreType.DMA,
            pltpu.SemaphoreType.DMA,
            pltpu.SemaphoreType.DMA,
            pltpu.SemaphoreType.DMA,
            pltpu.SemaphoreType.DMA,
            pltpu.SemaphoreType.DMA,
            pltpu.SemaphoreType.DMA,
        ])
    def gk(hf, rowp, colp, fr, fc, rowb, colb,
           bra, bca, brb, bcb,
           gra, gca, grb, gcb, wra, wca, wrb, wcb):
        c = lax.axis_index("c")
        s = lax.axis_index("s")
        w = c * 16 + s

        def outer(o, carry):
            eb = w * T + o * OCH

            @pl.when(o > 0)
            def _():
                pltpu.make_async_copy(bra, fr.at[pl.ds(eb, C)], wra).wait()
                pltpu.make_async_copy(bca, fc.at[pl.ds(eb, C)], wca).wait()
                pltpu.make_async_copy(brb, fr.at[pl.ds(eb, C)], wrb).wait()
                pltpu.make_async_copy(bcb, fc.at[pl.ds(eb, C)], wcb).wait()

            pltpu.sync_copy(rowp.at[pl.ds(eb, OCH)], rowb)
            pltpu.sync_copy(colp.at[pl.ds(eb, OCH)], colb)
            pltpu.async_copy(hf.at[rowb.at[pl.ds(0, C)]], bra, gra)
            pltpu.async_copy(hf.at[colb.at[pl.ds(0, C)]], bca, gca)
            pltpu.async_copy(hf.at[rowb.at[pl.ds(C, C)]], brb, grb)
            pltpu.async_copy(hf.at[colb.at[pl.ds(C, C)]], bcb, gcb)
            pltpu.make_async_copy(hf.at[rowb.at[pl.ds(0, C)]], bra, gra).wait()
            pltpu.async_copy(bra, fr.at[pl.ds(eb, C)], wra)
            pltpu.make_async_copy(hf.at[colb.at[pl.ds(0, C)]], bca, gca).wait()
            pltpu.async_copy(bca, fc.at[pl.ds(eb, C)], wca)
            pltpu.make_async_copy(hf.at[rowb.at[pl.ds(C, C)]], brb, grb).wait()
            pltpu.async_copy(brb, fr.at[pl.ds(eb + C, C)], wrb)
            pltpu.make_async_copy(hf.at[colb.at[pl.ds(C, C)]], bcb, gcb).wait()
            pltpu.async_copy(bcb, fc.at[pl.ds(eb + C, C)], wcb)
            return carry

        lax.fori_loop(0, NO, outer, 0)
        pltpu.make_async_copy(bra, fr.at[pl.ds(0, C)], wra).wait()
        pltpu.make_async_copy(bca, fc.at[pl.ds(0, C)], wca).wait()
        pltpu.make_async_copy(brb, fr.at[pl.ds(0, C)], wrb).wait()
        pltpu.make_async_copy(bcb, fc.at[pl.ds(0, C)], wcb).wait()

    return gk


_sc64 = _make_sc64()
_scgather = _make_scgather()


def _edge_stats(fr, fc):
    BR = 2000
    NB = E // BR

    def body(frr, fcr, so, xo, ssum, sxtx):
        i = pl.program_id(0)
        e = jnp.concatenate([frr[...], fcr[...]], axis=1)

        @pl.when(i == 0)
        def _():
            ssum[...] = jnp.zeros_like(ssum)
            sxtx[...] = jnp.zeros_like(sxtx)

        ssum[...] += jnp.sum(e, axis=0, keepdims=True)
        sxtx[...] += lax.dot_general(e, e, (((0,), (0,)), ((), ())),
                                     preferred_element_type=F32)

        @pl.when(i == NB - 1)
        def _():
            so[...] = ssum[...]
            xo[...] = sxtx[...]

    rowspec = pl.BlockSpec((BR, OUT), lambda i: (i, 0))
    return pl.pallas_call(
        body,
        grid=(NB,),
        in_specs=[rowspec, rowspec],
        out_specs=[pl.BlockSpec((1, 32), lambda i: (0, 0)),
                   pl.BlockSpec((32, 32), lambda i: (0, 0))],
        out_shape=[jax.ShapeDtypeStruct((1, 32), F32),
                   jax.ShapeDtypeStruct((32, 32), F32)],
        scratch_shapes=[pltpu.VMEM((1, 32), F32), pltpu.VMEM((32, 32), F32)],
    )(fr, fc)


def _edge_mlp(fr, fc, Ws1, t1, Ws2, t2, mW2, mb2):
    BR = 2048
    NB = EP // BR

    def body(frr, fcr, w1r, t1r, w2r, t2r, mwr, mbr, out_ref):
        e = jnp.concatenate([frr[...], fcr[...]], axis=1)
        a1 = jnp.maximum(jnp.dot(e, w1r[...], preferred_element_type=F32)
                         + t1r[...], 0.0)
        a2 = jnp.maximum(jnp.dot(e, w2r[...], preferred_element_type=F32)
                         + t2r[...], 0.0)
        p = (jnp.dot(a1, mwr[...], preferred_element_type=F32)
             + jnp.dot(a2, mwr[...], preferred_element_type=F32)) * 0.5
        p = p + mbr[...]
        out_ref[...] = (1.0 / (1.0 + jnp.exp(-p))).reshape(BR)

    rowspec = pl.BlockSpec((BR, OUT), lambda i: (i, 0))

    def _full(shape):
        return pl.BlockSpec(shape, lambda i: tuple(0 for _ in shape))

    return pl.pallas_call(
        body,
        grid=(NB,),
        in_specs=[rowspec, rowspec, _full(Ws1.shape), _full(t1.shape),
                  _full(Ws2.shape), _full(t2.shape), _full(mW2.shape),
                  _full(mb2.shape)],
        out_specs=pl.BlockSpec((BR,), lambda i: (i,)),
        out_shape=jax.ShapeDtypeStruct((EP,), F32),
    )(fr, fc, Ws1, t1, Ws2, t2, mW2, mb2)


def kernel(x, edge_index, edge_weight, Ws, bs, bn_g, bn_b, fin_g, fin_b,
           mW1, mb1, mg, mbt, mW2, mb2):
    row = edge_index[0].astype(I32)
    col = edge_index[1].astype(I32)
    npad = EP - E
    it = jnp.arange(npad, dtype=I32)
    rowp = jnp.concatenate([row, it % 8192])
    colp = jnp.concatenate([col, N + (it % 1024)])
    wp = jnp.concatenate([edge_weight, jnp.zeros((npad,), F32)])
    col2 = colp.reshape(EP // 256, 256)
    z128 = jnp.zeros((128, 32), F32)

    xp = jnp.zeros((NP, 8), F32).at[:N, :2].set(x)
    W0 = jnp.zeros((8, HID), F32).at[:2].set(Ws[0])
    g0 = jnp.ones((1, 8), F32).at[0, :2].set(bn_g[0])
    bb0 = jnp.zeros((1, 8), F32).at[0, :2].set(bn_b[0])
    hlo, hhi = _dense([xp], W0, g0, bb0, bs[0].reshape(1, HID),
                      relu_in=False, out_widths=(32, 32))

    W11 = jnp.zeros((HID, HID), F32).at[:, :OUT].set(Ws[11])
    b11 = jnp.zeros((HID,), F32).at[:OUT].set(bs[11])
    Wst = jnp.stack(Ws[1:11] + [W11, Ws[1]])
    bst = jnp.stack(bs[1:11] + [b11, bs[1]]).reshape(12, 1, HID)
    gst = jnp.stack(bn_g[1:12] + [bn_g[1]]).reshape(12, 1, HID)
    nst = jnp.stack(bn_b[1:12] + [bn_b[1]]).reshape(12, 1, HID)

    def step(carry, ws):
        tlo, thi, _, _ = carry
        W, b, g, bb = ws
        rl, rh = _sc64(tlo, thi, rowp, col2, wp, z128)
        tlo2, thi2 = _dense([rl, rh], W, g, bb, b,
                            relu_in=True, out_widths=(32, 32))
        return (tlo2, thi2, rl, rh), None

    (_, _, rl, rh), _ = lax.scan(step, (hlo, hhi, hlo, hhi),
                                 (Wst, bst, gst, nst))
    hf = _hf_apply(rl, rh, fin_g.reshape(1, OUT), fin_b.reshape(1, OUT))

    fr, fc = _scgather(hf, rowp, colp)
    sums, xtx = _edge_stats(fr, fc)
    mu_e = sums.reshape(32) / E
    C = xtx / E - jnp.outer(mu_e, mu_e)
    W1r = jnp.roll(mW1, OUT, axis=0)

    def fold(W1):
        mu_z = mu_e @ W1 + mb1
        var_z = jnp.sum(W1 * (C @ W1), axis=0)
        s = mg / jnp.sqrt(var_z + EPS)
        t = (mb1 - mu_z) * s + mbt
        return W1 * s[None, :], t.reshape(1, MLP_HID)

    Ws1, t1 = fold(mW1)
    Ws2, t2 = fold(W1r)
    out = _edge_mlp(fr, fc, Ws1, t1, Ws2, t2, mW2, mb2.reshape(1, 1))
    return out[:E]


# speedup vs baseline: 6.2019x; 1.0091x over previous
import functools

import jax
import jax.numpy as jnp
from jax import lax
from jax.experimental import pallas as pl
from jax.experimental.pallas import tpu as pltpu
from jax.experimental.pallas import tpu_sc as plsc

N = 50000
E = 800000
NP = 51200
EP = 819200
HID = 64
OUT = 16
MLP_HID = 256
NREAL = float(N)
EPS = 1e-5

F32 = jnp.float32
I32 = jnp.int32


def _dense(parts, W, g, beta, bias, relu_in, out_widths):
    n_parts = len(parts)
    pw = [int(p.shape[1]) for p in parts]
    FW = sum(pw)
    DBR = 2048
    NB = NP // DBR

    def body(*refs):
        hrefs = refs[:n_parts]
        Wr, gr, br, bir = refs[n_parts:n_parts + 4]
        outs = refs[n_parts + 4:-1]
        st = refs[-1]
        ph = pl.program_id(0)
        i = pl.program_id(1)
        if n_parts > 1:
            a = jnp.concatenate([r[...] for r in hrefs], axis=1)
        else:
            a = hrefs[0][...]
        if relu_in:
            a = jnp.maximum(a, 0.0)

        @pl.when(ph == 0)
        def _():
            @pl.when(i == 0)
            def _():
                st[...] = jnp.zeros_like(st)
            st[0:1, :] += jnp.sum(a, axis=0, keepdims=True)
            st[1:2, :] += jnp.sum(a * a, axis=0, keepdims=True)

        @pl.when(ph == 1)
        def _():
            mu = st[0:1, :] / NREAL
            var = st[1:2, :] / NREAL - mu * mu
            sc = gr[...] * lax.rsqrt(var + EPS)
            t = br[...] - mu * sc
            y = jnp.dot(a * sc + t, Wr[...], preferred_element_type=F32)
            y = y + bir[...]
            off = 0
            for oref, ow in zip(outs, out_widths):
                oref[...] = y[:, off:off + ow]
                off += ow

    def _rowspec(w_):
        return pl.BlockSpec((DBR, w_), lambda p, i: (i, 0))

    def _full(shape):
        return pl.BlockSpec(shape, lambda p, i: (0, 0))

    in_specs = [_rowspec(w_) for w_ in pw] + [
        _full(W.shape), _full(g.shape), _full(beta.shape), _full(bias.shape)]
    out_specs = [_rowspec(ow) for ow in out_widths]
    out_shape = [jax.ShapeDtypeStruct((NP, ow), F32) for ow in out_widths]
    return pl.pallas_call(
        body,
        grid=(2, NB),
        in_specs=in_specs,
        out_specs=out_specs,
        out_shape=out_shape,
        scratch_shapes=[pltpu.VMEM((2, FW), F32)],
    )(*parts, W, g, beta, bias)


def _hf_apply(p0, p1, g, beta):
    DBR = 2048
    NB = NP // DBR

    def body(a_ref, b_ref, gr, br, out_ref, st):
        ph = pl.program_id(0)
        i = pl.program_id(1)
        a = jnp.maximum(a_ref[...] + b_ref[...], 0.0)[:, :OUT]

        @pl.when(ph == 0)
        def _():
            @pl.when(i == 0)
            def _():
                st[...] = jnp.zeros_like(st)
            st[0:1, :] += jnp.sum(a, axis=0, keepdims=True)
            st[1:2, :] += jnp.sum(a * a, axis=0, keepdims=True)

        @pl.when(ph == 1)
        def _():
            mu = st[0:1, :] / NREAL
            var = st[1:2, :] / NREAL - mu * mu
            sc = gr[...] * lax.rsqrt(var + EPS)
            t = br[...] - mu * sc
            out_ref[...] = a * sc + t

    inspec = pl.BlockSpec((DBR, 32), lambda p, i: (i, 0))
    rowspec = pl.BlockSpec((DBR, OUT), lambda p, i: (i, 0))
    full = pl.BlockSpec((1, OUT), lambda p, i: (0, 0))
    return pl.pallas_call(
        body,
        grid=(2, NB),
        in_specs=[inspec, inspec, full, full],
        out_specs=rowspec,
        out_shape=jax.ShapeDtypeStruct((NP, OUT), F32),
        scratch_shapes=[pltpu.VMEM((2, OUT), F32)],
    )(p0, p1, g, beta)


def _make_sc64():
    mesh = plsc.VectorSubcoreMesh(core_axis_name="c", subcore_axis_name="s")
    T = EP // 16
    BIG = 2560
    C = 256
    NK = BIG // C
    NO = T // BIG
    RPT = NP // 16

    @functools.partial(
        pl.kernel, mesh=mesh,
        compiler_params=pltpu.CompilerParams(use_tc_tiling_on_sc=False),
        out_type=(jax.ShapeDtypeStruct((NP, 32), F32),
                  jax.ShapeDtypeStruct((NP, 32), F32)),
        scratch_types=[
            pltpu.VMEM((BIG,), I32),
            pltpu.VMEM((NK, C), I32),
            pltpu.VMEM((BIG,), F32),
            pltpu.VMEM((C, 32), F32),
            pltpu.VMEM((C, 32), F32),
            pltpu.VMEM((128, 32), F32),
            pltpu.VMEM_SHARED((NP, 32), F32),
            pltpu.SemaphoreType.DMA,
            pltpu.SemaphoreType.DMA,
            pltpu.SemaphoreType.DMA,
            pltpu.SemaphoreType.DMA,
        ])
    def sck(hlo, hhi, rowp, col2, wp, z128,
            out_lo, out_hi, rowb, colb, wb, gbufa, gbufb, zbuf, acc,
            gsema, gsemb, ssema, ssemb):
        c = lax.axis_index("c")
        s = lax.axis_index("s")
        pltpu.sync_copy(z128, zbuf)
        for j in range(RPT // 128):
            pltpu.sync_copy(zbuf, acc.at[pl.ds(s * RPT + j * 128, 128)])
        plsc.subcore_barrier()

        gbufs = (gbufa, gbufb)
        gsems = (gsema, gsemb)
        ssems = (ssema, ssemb)

        def sdrain(k, gbuf, ssem):
            pltpu.make_async_copy(gbuf, acc.at[colb.at[k]], ssem).wait()

        def gissue(k, gbuf, gsem):
            @pl.when(c == 0)
            def _():
                pltpu.async_copy(hlo.at[rowb.at[pl.ds(k * C, C)]], gbuf, gsem)

            @pl.when(c == 1)
            def _():
                pltpu.async_copy(hhi.at[rowb.at[pl.ds(k * C, C)]], gbuf, gsem)

        def outer(o, carry):
            @pl.when(o > 0)
            def _():
                sdrain(NK - 2, gbufs[0], ssems[0])
                sdrain(NK - 1, gbufs[1], ssems[1])

            eb = s * T + o * BIG
            pltpu.sync_copy(rowp.at[pl.ds(eb, BIG)], rowb)
            pltpu.sync_copy(col2.at[pl.ds(eb // C, NK)], colb)
            pltpu.sync_copy(wp.at[pl.ds(eb, BIG)], wb)
            gissue(0, gbufs[0], gsems[0])
            for k in range(NK):
                gbuf, gsem = gbufs[k % 2], gsems[k % 2]
                if k + 1 < NK:
                    if k + 1 >= 2:
                        sdrain(k - 1, gbufs[(k + 1) % 2], ssems[(k + 1) % 2])
                    gissue(k + 1, gbufs[(k + 1) % 2], gsems[(k + 1) % 2])
                pltpu.make_async_copy(
                    hlo.at[rowb.at[pl.ds(k * C, C)]], gbuf, gsem).wait()

                def scale(g, cc):
                    wvec = wb[pl.ds(k * C + g * 16, 16)]
                    for kk in range(16):
                        e = g * 16 + kk
                        wv = wvec[kk]
                        gbuf[e, pl.ds(0, 16)] = gbuf[e, pl.ds(0, 16)] * wv
                        gbuf[e, pl.ds(16, 16)] = gbuf[e, pl.ds(16, 16)] * wv
                    return cc

                lax.fori_loop(0, C // 16, scale, 0)
                pltpu.async_copy(gbuf, acc.at[colb.at[k]], ssems[k % 2],
                                 add=True)
            return carry

        lax.fori_loop(0, NO, outer, 0)
        sdrain(NK - 2, gbufs[0], ssems[0])
        sdrain(NK - 1, gbufs[1], ssems[1])
        plsc.subcore_barrier()

        @pl.when(c == 0)
        def _():
            pltpu.sync_copy(acc.at[pl.ds(s * RPT, RPT)],
                            out_lo.at[pl.ds(s * RPT, RPT)])

        @pl.when(c == 1)
        def _():
            pltpu.sync_copy(acc.at[pl.ds(s * RPT, RPT)],
                            out_hi.at[pl.ds(s * RPT, RPT)])

    return sck


def _make_scgather():
    mesh = plsc.VectorSubcoreMesh(core_axis_name="c", subcore_axis_name="s")
    T = EP // 32
    OCH = 2560
    C = 1280
    NO = T // OCH

    @functools.partial(
        pl.kernel, mesh=mesh,
        compiler_params=pltpu.CompilerParams(use_tc_tiling_on_sc=False),
        out_type=(jax.ShapeDtypeStruct((EP, OUT), F32),
                  jax.ShapeDtypeStruct((EP, OUT), F32)),
        scratch_types=[
            pltpu.VMEM((OCH,), I32),
            pltpu.VMEM((OCH,), I32),
            pltpu.VMEM((C, OUT), F32),
            pltpu.VMEM((C, OUT), F32),
            pltpu.VMEM((C, OUT), F32),
            pltpu.VMEM((C, OUT), F32),
            pltpu.SemaphoreType.DMA,
            pltpu.SemaphoreType.DMA,
            pltpu.SemaphoreType.DMA,
            pltpu.SemaphoreType.DMA,
            pltpu.SemaphoreType.DMA,
            pltpu.SemaphoreType.DMA,
            pltpu.SemaphoreType.DMA,
            pltpu.SemaphoreType.DMA,
        ])
    def gk(hf, rowp, colp, fr, fc, rowb, colb,
           bra, bca, brb, bcb,
           gra, gca, grb, gcb, wra, wca, wrb, wcb):
        c = lax.axis_index("c")
        s = lax.axis_index("s")
        w = c * 16 + s

        def outer(o, carry):
            eb = w * T + o * OCH

            @pl.when(o > 0)
            def _():
                pltpu.make_async_copy(bra, fr.at[pl.ds(eb, C)], wra).wait()
                pltpu.make_async_copy(bca, fc.at[pl.ds(eb, C)], wca).wait()
                pltpu.make_async_copy(brb, fr.at[pl.ds(eb, C)], wrb).wait()
                pltpu.make_async_copy(bcb, fc.at[pl.ds(eb, C)], wcb).wait()

            pltpu.sync_copy(rowp.at[pl.ds(eb, OCH)], rowb)
            pltpu.sync_copy(colp.at[pl.ds(eb, OCH)], colb)
            pltpu.async_copy(hf.at[rowb.at[pl.ds(0, C)]], bra, gra)
            pltpu.async_copy(hf.at[colb.at[pl.ds(0, C)]], bca, gca)
            pltpu.async_copy(hf.at[rowb.at[pl.ds(C, C)]], brb, grb)
            pltpu.async_copy(hf.at[colb.at[pl.ds(C, C)]], bcb, gcb)
            pltpu.make_async_copy(hf.at[rowb.at[pl.ds(0, C)]], bra, gra).wait()
            pltpu.async_copy(bra, fr.at[pl.ds(eb, C)], wra)
            pltpu.make_async_copy(hf.at[colb.at[pl.ds(0, C)]], bca, gca).wait()
            pltpu.async_copy(bca, fc.at[pl.ds(eb, C)], wca)
            pltpu.make_async_copy(hf.at[rowb.at[pl.ds(C, C)]], brb, grb).wait()
            pltpu.async_copy(brb, fr.at[pl.ds(eb + C, C)], wrb)
            pltpu.make_async_copy(hf.at[colb.at[pl.ds(C, C)]], bcb, gcb).wait()
            pltpu.async_copy(bcb, fc.at[pl.ds(eb + C, C)], wcb)
            return carry

        lax.fori_loop(0, NO, outer, 0)
        pltpu.make_async_copy(bra, fr.at[pl.ds(0, C)], wra).wait()
        pltpu.make_async_copy(bca, fc.at[pl.ds(0, C)], wca).wait()
        pltpu.make_async_copy(brb, fr.at[pl.ds(0, C)], wrb).wait()
        pltpu.make_async_copy(bcb, fc.at[pl.ds(0, C)], wcb).wait()

    return gk


_sc64 = _make_sc64()
_scgather = _make_scgather()


def _edge_stats(fr, fc):
    BR = 2000
    NB = E // BR

    def body(frr, fcr, so, xo, ssum, sxtx):
        i = pl.program_id(0)
        e = jnp.concatenate([frr[...], fcr[...]], axis=1)

        @pl.when(i == 0)
        def _():
            ssum[...] = jnp.zeros_like(ssum)
            sxtx[...] = jnp.zeros_like(sxtx)

        ssum[...] += jnp.sum(e, axis=0, keepdims=True)
        sxtx[...] += lax.dot_general(e, e, (((0,), (0,)), ((), ())),
                                     preferred_element_type=F32)

        @pl.when(i == NB - 1)
        def _():
            so[...] = ssum[...]
            xo[...] = sxtx[...]

    rowspec = pl.BlockSpec((BR, OUT), lambda i: (i, 0))
    return pl.pallas_call(
        body,
        grid=(NB,),
        in_specs=[rowspec, rowspec],
        out_specs=[pl.BlockSpec((1, 32), lambda i: (0, 0)),
                   pl.BlockSpec((32, 32), lambda i: (0, 0))],
        out_shape=[jax.ShapeDtypeStruct((1, 32), F32),
                   jax.ShapeDtypeStruct((32, 32), F32)],
        scratch_shapes=[pltpu.VMEM((1, 32), F32), pltpu.VMEM((32, 32), F32)],
    )(fr, fc)


def _edge_mlp(fr, fc, Ws1, t1, Ws2, t2, mW2, mb2):
    BR = 2048
    NB = EP // BR

    def body(frr, fcr, w1r, t1r, w2r, t2r, mwr, mbr, out_ref):
        e = jnp.concatenate([frr[...], fcr[...]], axis=1).astype(jnp.bfloat16)
        a1 = jnp.maximum(jnp.dot(e, w1r[...], preferred_element_type=F32)
                         + t1r[...], 0.0)
        a2 = jnp.maximum(jnp.dot(e, w2r[...], preferred_element_type=F32)
                         + t2r[...], 0.0)
        p = jnp.dot((a1 + a2).astype(jnp.bfloat16), mwr[...],
                    preferred_element_type=F32) * 0.5
        p = p + mbr[...]
        out_ref[...] = (1.0 / (1.0 + jnp.exp(-p))).reshape(BR)

    rowspec = pl.BlockSpec((BR, OUT), lambda i: (i, 0))

    def _full(shape):
        return pl.BlockSpec(shape, lambda i: tuple(0 for _ in shape))

    return pl.pallas_call(
        body,
        grid=(NB,),
        in_specs=[rowspec, rowspec, _full(Ws1.shape), _full(t1.shape),
                  _full(Ws2.shape), _full(t2.shape), _full(mW2.shape),
                  _full(mb2.shape)],
        out_specs=pl.BlockSpec((BR,), lambda i: (i,)),
        out_shape=jax.ShapeDtypeStruct((EP,), F32),
    )(fr, fc, Ws1, t1, Ws2, t2, mW2, mb2)


def kernel(x, edge_index, edge_weight, Ws, bs, bn_g, bn_b, fin_g, fin_b,
           mW1, mb1, mg, mbt, mW2, mb2):
    row = edge_index[0].astype(I32)
    col = edge_index[1].astype(I32)
    npad = EP - E
    it = jnp.arange(npad, dtype=I32)
    rowp = jnp.concatenate([row, it % 8192])
    colp = jnp.concatenate([col, N + (it % 1024)])
    wp = jnp.concatenate([edge_weight, jnp.zeros((npad,), F32)])
    col2 = colp.reshape(EP // 256, 256)
    z128 = jnp.zeros((128, 32), F32)

    xp = jnp.zeros((NP, 8), F32).at[:N, :2].set(x)
    W0 = jnp.zeros((8, HID), F32).at[:2].set(Ws[0])
    g0 = jnp.ones((1, 8), F32).at[0, :2].set(bn_g[0])
    bb0 = jnp.zeros((1, 8), F32).at[0, :2].set(bn_b[0])
    hlo, hhi = _dense([xp], W0, g0, bb0, bs[0].reshape(1, HID),
                      relu_in=False, out_widths=(32, 32))

    W11 = jnp.zeros((HID, HID), F32).at[:, :OUT].set(Ws[11])
    b11 = jnp.zeros((HID,), F32).at[:OUT].set(bs[11])
    Wst = jnp.stack(Ws[1:11] + [W11, Ws[1]])
    bst = jnp.stack(bs[1:11] + [b11, bs[1]]).reshape(12, 1, HID)
    gst = jnp.stack(bn_g[1:12] + [bn_g[1]]).reshape(12, 1, HID)
    nst = jnp.stack(bn_b[1:12] + [bn_b[1]]).reshape(12, 1, HID)

    def step(carry, ws):
        tlo, thi, _, _ = carry
        W, b, g, bb = ws
        rl, rh = _sc64(tlo, thi, rowp, col2, wp, z128)
        tlo2, thi2 = _dense([rl, rh], W, g, bb, b,
                            relu_in=True, out_widths=(32, 32))
        return (tlo2, thi2, rl, rh), None

    (_, _, rl, rh), _ = lax.scan(step, (hlo, hhi, hlo, hhi),
                                 (Wst, bst, gst, nst))
    hf = _hf_apply(rl, rh, fin_g.reshape(1, OUT), fin_b.reshape(1, OUT))

    fr, fc = _scgather(hf, rowp, colp)
    sums, xtx = _edge_stats(fr, fc)
    mu_e = sums.reshape(32) / E
    C = xtx / E - jnp.outer(mu_e, mu_e)
    W1r = jnp.roll(mW1, OUT, axis=0)

    def fold(W1):
        mu_z = mu_e @ W1 + mb1
        var_z = jnp.sum(W1 * (C @ W1), axis=0)
        s = mg / jnp.sqrt(var_z + EPS)
        t = (mb1 - mu_z) * s + mbt
        return W1 * s[None, :], t.reshape(1, MLP_HID)

    Ws1, t1 = fold(mW1)
    Ws2, t2 = fold(W1r)
    out = _edge_mlp(fr, fc, Ws1.astype(jnp.bfloat16), t1,
                    Ws2.astype(jnp.bfloat16), t2,
                    mW2.astype(jnp.bfloat16), mb2.reshape(1, 1))
    return out[:E]


# speedup vs baseline: 6.2088x; 1.0011x over previous
import functools

import jax
import jax.numpy as jnp
from jax import lax
from jax.experimental import pallas as pl
from jax.experimental.pallas import tpu as pltpu
from jax.experimental.pallas import tpu_sc as plsc

N = 50000
E = 800000
NP = 51200
EP = 819200
HID = 64
OUT = 16
MLP_HID = 256
NREAL = float(N)
EPS = 1e-5

F32 = jnp.float32
I32 = jnp.int32


def _dense(parts, W, g, beta, bias, relu_in, out_widths):
    n_parts = len(parts)
    pw = [int(p.shape[1]) for p in parts]
    FW = sum(pw)
    DBR = 2048
    NB = NP // DBR

    def body(*refs):
        hrefs = refs[:n_parts]
        Wr, gr, br, bir = refs[n_parts:n_parts + 4]
        outs = refs[n_parts + 4:-1]
        st = refs[-1]
        ph = pl.program_id(0)
        i = pl.program_id(1)
        if n_parts > 1:
            a = jnp.concatenate([r[...] for r in hrefs], axis=1)
        else:
            a = hrefs[0][...]
        if relu_in:
            a = jnp.maximum(a, 0.0)

        @pl.when(ph == 0)
        def _():
            @pl.when(i == 0)
            def _():
                st[...] = jnp.zeros_like(st)
            st[0:1, :] += jnp.sum(a, axis=0, keepdims=True)
            st[1:2, :] += jnp.sum(a * a, axis=0, keepdims=True)

        @pl.when(ph == 1)
        def _():
            mu = st[0:1, :] / NREAL
            var = st[1:2, :] / NREAL - mu * mu
            sc = gr[...] * lax.rsqrt(var + EPS)
            t = br[...] - mu * sc
            y = jnp.dot(a * sc + t, Wr[...], preferred_element_type=F32)
            y = y + bir[...]
            off = 0
            for oref, ow in zip(outs, out_widths):
                oref[...] = y[:, off:off + ow]
                off += ow

    def _rowspec(w_):
        return pl.BlockSpec((DBR, w_), lambda p, i: (i, 0))

    def _full(shape):
        return pl.BlockSpec(shape, lambda p, i: (0, 0))

    in_specs = [_rowspec(w_) for w_ in pw] + [
        _full(W.shape), _full(g.shape), _full(beta.shape), _full(bias.shape)]
    out_specs = [_rowspec(ow) for ow in out_widths]
    out_shape = [jax.ShapeDtypeStruct((NP, ow), F32) for ow in out_widths]
    return pl.pallas_call(
        body,
        grid=(2, NB),
        in_specs=in_specs,
        out_specs=out_specs,
        out_shape=out_shape,
        scratch_shapes=[pltpu.VMEM((2, FW), F32)],
    )(*parts, W, g, beta, bias)


def _hf_apply(p0, p1, g, beta):
    DBR = 2048
    NB = NP // DBR

    def body(a_ref, b_ref, gr, br, out_ref, st):
        ph = pl.program_id(0)
        i = pl.program_id(1)
        a = jnp.maximum(a_ref[...] + b_ref[...], 0.0)[:, :OUT]

        @pl.when(ph == 0)
        def _():
            @pl.when(i == 0)
            def _():
                st[...] = jnp.zeros_like(st)
            st[0:1, :] += jnp.sum(a, axis=0, keepdims=True)
            st[1:2, :] += jnp.sum(a * a, axis=0, keepdims=True)

        @pl.when(ph == 1)
        def _():
            mu = st[0:1, :] / NREAL
            var = st[1:2, :] / NREAL - mu * mu
            sc = gr[...] * lax.rsqrt(var + EPS)
            t = br[...] - mu * sc
            out_ref[...] = a * sc + t

    inspec = pl.BlockSpec((DBR, 32), lambda p, i: (i, 0))
    rowspec = pl.BlockSpec((DBR, OUT), lambda p, i: (i, 0))
    full = pl.BlockSpec((1, OUT), lambda p, i: (0, 0))
    return pl.pallas_call(
        body,
        grid=(2, NB),
        in_specs=[inspec, inspec, full, full],
        out_specs=rowspec,
        out_shape=jax.ShapeDtypeStruct((NP, OUT), F32),
        scratch_shapes=[pltpu.VMEM((2, OUT), F32)],
    )(p0, p1, g, beta)


def _make_sc64():
    mesh = plsc.VectorSubcoreMesh(core_axis_name="c", subcore_axis_name="s")
    T = EP // 16
    BIG = 2560
    C = 256
    NK = BIG // C
    NO = T // BIG
    RPT = NP // 16

    @functools.partial(
        pl.kernel, mesh=mesh,
        compiler_params=pltpu.CompilerParams(use_tc_tiling_on_sc=False),
        out_type=(jax.ShapeDtypeStruct((NP, 32), F32),
                  jax.ShapeDtypeStruct((NP, 32), F32)),
        scratch_types=[
            pltpu.VMEM((BIG,), I32),
            pltpu.VMEM((NK, C), I32),
            pltpu.VMEM((BIG,), F32),
            pltpu.VMEM((C, 32), F32),
            pltpu.VMEM((C, 32), F32),
            pltpu.VMEM((128, 32), F32),
            pltpu.VMEM_SHARED((NP, 32), F32),
            pltpu.SemaphoreType.DMA,
            pltpu.SemaphoreType.DMA,
            pltpu.SemaphoreType.DMA,
            pltpu.SemaphoreType.DMA,
        ])
    def sck(hlo, hhi, rowp, col2, wp, z128,
            out_lo, out_hi, rowb, colb, wb, gbufa, gbufb, zbuf, acc,
            gsema, gsemb, ssema, ssemb):
        c = lax.axis_index("c")
        s = lax.axis_index("s")
        pltpu.sync_copy(z128, zbuf)
        for j in range(RPT // 128):
            pltpu.sync_copy(zbuf, acc.at[pl.ds(s * RPT + j * 128, 128)])
        plsc.subcore_barrier()

        gbufs = (gbufa, gbufb)
        gsems = (gsema, gsemb)
        ssems = (ssema, ssemb)

        def sdrain(k, gbuf, ssem):
            pltpu.make_async_copy(gbuf, acc.at[colb.at[k]], ssem).wait()

        def gissue(k, gbuf, gsem):
            @pl.when(c == 0)
            def _():
                pltpu.async_copy(hlo.at[rowb.at[pl.ds(k * C, C)]], gbuf, gsem)

            @pl.when(c == 1)
            def _():
                pltpu.async_copy(hhi.at[rowb.at[pl.ds(k * C, C)]], gbuf, gsem)

        def outer(o, carry):
            @pl.when(o > 0)
            def _():
                sdrain(NK - 2, gbufs[0], ssems[0])
                sdrain(NK - 1, gbufs[1], ssems[1])

            eb = s * T + o * BIG
            pltpu.sync_copy(rowp.at[pl.ds(eb, BIG)], rowb)
            pltpu.sync_copy(col2.at[pl.ds(eb // C, NK)], colb)
            pltpu.sync_copy(wp.at[pl.ds(eb, BIG)], wb)
            gissue(0, gbufs[0], gsems[0])
            for k in range(NK):
                gbuf, gsem = gbufs[k % 2], gsems[k % 2]
                if k + 1 < NK:
                    if k + 1 >= 2:
                        sdrain(k - 1, gbufs[(k + 1) % 2], ssems[(k + 1) % 2])
                    gissue(k + 1, gbufs[(k + 1) % 2], gsems[(k + 1) % 2])
                pltpu.make_async_copy(
                    hlo.at[rowb.at[pl.ds(k * C, C)]], gbuf, gsem).wait()

                def scale(g, cc):
                    wvec = wb[pl.ds(k * C + g * 16, 16)]
                    for kk in range(16):
                        e = g * 16 + kk
                        wv = wvec[kk]
                        gbuf[e, pl.ds(0, 16)] = gbuf[e, pl.ds(0, 16)] * wv
                        gbuf[e, pl.ds(16, 16)] = gbuf[e, pl.ds(16, 16)] * wv
                    return cc

                lax.fori_loop(0, C // 16, scale, 0)
                pltpu.async_copy(gbuf, acc.at[colb.at[k]], ssems[k % 2],
                                 add=True)
            return carry

        lax.fori_loop(0, NO, outer, 0)
        sdrain(NK - 2, gbufs[0], ssems[0])
        sdrain(NK - 1, gbufs[1], ssems[1])
        plsc.subcore_barrier()

        @pl.when(c == 0)
        def _():
            pltpu.sync_copy(acc.at[pl.ds(s * RPT, RPT)],
                            out_lo.at[pl.ds(s * RPT, RPT)])

        @pl.when(c == 1)
        def _():
            pltpu.sync_copy(acc.at[pl.ds(s * RPT, RPT)],
                            out_hi.at[pl.ds(s * RPT, RPT)])

    return sck


def _make_scgather():
    mesh = plsc.VectorSubcoreMesh(core_axis_name="c", subcore_axis_name="s")
    T = EP // 32
    OCH = 2560
    C = 1280
    NO = T // OCH

    @functools.partial(
        pl.kernel, mesh=mesh,
        compiler_params=pltpu.CompilerParams(use_tc_tiling_on_sc=False),
        out_type=(jax.ShapeDtypeStruct((EP, OUT), F32),
                  jax.ShapeDtypeStruct((EP, OUT), F32)),
        scratch_types=[
            pltpu.VMEM((OCH,), I32),
            pltpu.VMEM((OCH,), I32),
            pltpu.VMEM((C, OUT), F32),
            pltpu.VMEM((C, OUT), F32),
            pltpu.VMEM((C, OUT), F32),
            pltpu.VMEM((C, OUT), F32),
            pltpu.SemaphoreType.DMA,
            pltpu.SemaphoreType.DMA,
            pltpu.SemaphoreType.DMA,
            pltpu.SemaphoreType.DMA,
            pltpu.SemaphoreType.DMA,
            pltpu.SemaphoreType.DMA,
            pltpu.SemaphoreType.DMA,
            pltpu.SemaphoreType.DMA,
        ])
    def gk(hf, rowp, colp, fr, fc, rowb, colb,
           bra, bca, brb, bcb,
           gra, gca, grb, gcb, wra, wca, wrb, wcb):
        c = lax.axis_index("c")
        s = lax.axis_index("s")
        w = c * 16 + s

        def outer(o, carry):
            eb = w * T + o * OCH

            @pl.when(o > 0)
            def _():
                pltpu.make_async_copy(bra, fr.at[pl.ds(eb, C)], wra).wait()
                pltpu.make_async_copy(bca, fc.at[pl.ds(eb, C)], wca).wait()
                pltpu.make_async_copy(brb, fr.at[pl.ds(eb, C)], wrb).wait()
                pltpu.make_async_copy(bcb, fc.at[pl.ds(eb, C)], wcb).wait()

            pltpu.sync_copy(rowp.at[pl.ds(eb, OCH)], rowb)
            pltpu.sync_copy(colp.at[pl.ds(eb, OCH)], colb)
            pltpu.async_copy(hf.at[rowb.at[pl.ds(0, C)]], bra, gra)
            pltpu.async_copy(hf.at[colb.at[pl.ds(0, C)]], bca, gca)
            pltpu.async_copy(hf.at[rowb.at[pl.ds(C, C)]], brb, grb)
            pltpu.async_copy(hf.at[colb.at[pl.ds(C, C)]], bcb, gcb)
            pltpu.make_async_copy(hf.at[rowb.at[pl.ds(0, C)]], bra, gra).wait()
            pltpu.async_copy(bra, fr.at[pl.ds(eb, C)], wra)
            pltpu.make_async_copy(hf.at[colb.at[pl.ds(0, C)]], bca, gca).wait()
            pltpu.async_copy(bca, fc.at[pl.ds(eb, C)], wca)
            pltpu.make_async_copy(hf.at[rowb.at[pl.ds(C, C)]], brb, grb).wait()
            pltpu.async_copy(brb, fr.at[pl.ds(eb + C, C)], wrb)
            pltpu.make_async_copy(hf.at[colb.at[pl.ds(C, C)]], bcb, gcb).wait()
            pltpu.async_copy(bcb, fc.at[pl.ds(eb + C, C)], wcb)
            return carry

        lax.fori_loop(0, NO, outer, 0)
        pltpu.make_async_copy(bra, fr.at[pl.ds(0, C)], wra).wait()
        pltpu.make_async_copy(bca, fc.at[pl.ds(0, C)], wca).wait()
        pltpu.make_async_copy(brb, fr.at[pl.ds(0, C)], wrb).wait()
        pltpu.make_async_copy(bcb, fc.at[pl.ds(0, C)], wcb).wait()

    return gk


_sc64 = _make_sc64()
_scgather = _make_scgather()


def _edge_stats(fr, fc):
    BR = 2000
    NB = E // BR

    def body(frr, fcr, so, xo, ssum, sxtx):
        i = pl.program_id(0)
        e = jnp.concatenate([frr[...], fcr[...]], axis=1)

        @pl.when(i == 0)
        def _():
            ssum[...] = jnp.zeros_like(ssum)
            sxtx[...] = jnp.zeros_like(sxtx)

        eb16 = e.astype(jnp.bfloat16)
        ssum[...] += jnp.sum(e, axis=0, keepdims=True)
        sxtx[...] += lax.dot_general(eb16, eb16, (((0,), (0,)), ((), ())),
                                     preferred_element_type=F32)

        @pl.when(i == NB - 1)
        def _():
            so[...] = ssum[...]
            xo[...] = sxtx[...]

    rowspec = pl.BlockSpec((BR, OUT), lambda i: (i, 0))
    return pl.pallas_call(
        body,
        grid=(NB,),
        in_specs=[rowspec, rowspec],
        out_specs=[pl.BlockSpec((1, 32), lambda i: (0, 0)),
                   pl.BlockSpec((32, 32), lambda i: (0, 0))],
        out_shape=[jax.ShapeDtypeStruct((1, 32), F32),
                   jax.ShapeDtypeStruct((32, 32), F32)],
        scratch_shapes=[pltpu.VMEM((1, 32), F32), pltpu.VMEM((32, 32), F32)],
    )(fr, fc)


def _edge_mlp(fr, fc, Ws1, t1, Ws2, t2, mW2, mb2):
    BR = 2048
    NB = EP // BR

    def body(frr, fcr, w1r, t1r, w2r, t2r, mwr, mbr, out_ref):
        e = jnp.concatenate([frr[...], fcr[...]], axis=1).astype(jnp.bfloat16)
        a1 = jnp.maximum(jnp.dot(e, w1r[...], preferred_element_type=F32)
                         + t1r[...], 0.0)
        a2 = jnp.maximum(jnp.dot(e, w2r[...], preferred_element_type=F32)
                         + t2r[...], 0.0)
        p = jnp.dot((a1 + a2).astype(jnp.bfloat16), mwr[...],
                    preferred_element_type=F32) * 0.5
        p = p + mbr[...]
        out_ref[...] = (1.0 / (1.0 + jnp.exp(-p))).reshape(BR)

    rowspec = pl.BlockSpec((BR, OUT), lambda i: (i, 0))

    def _full(shape):
        return pl.BlockSpec(shape, lambda i: tuple(0 for _ in shape))

    return pl.pallas_call(
        body,
        grid=(NB,),
        in_specs=[rowspec, rowspec, _full(Ws1.shape), _full(t1.shape),
                  _full(Ws2.shape), _full(t2.shape), _full(mW2.shape),
                  _full(mb2.shape)],
        out_specs=pl.BlockSpec((BR,), lambda i: (i,)),
        out_shape=jax.ShapeDtypeStruct((EP,), F32),
    )(fr, fc, Ws1, t1, Ws2, t2, mW2, mb2)


def kernel(x, edge_index, edge_weight, Ws, bs, bn_g, bn_b, fin_g, fin_b,
           mW1, mb1, mg, mbt, mW2, mb2):
    row = edge_index[0].astype(I32)
    col = edge_index[1].astype(I32)
    npad = EP - E
    it = jnp.arange(npad, dtype=I32)
    rowp = jnp.concatenate([row, it % 8192])
    colp = jnp.concatenate([col, N + (it % 1024)])
    wp = jnp.concatenate([edge_weight, jnp.zeros((npad,), F32)])
    col2 = colp.reshape(EP // 256, 256)
    z128 = jnp.zeros((128, 32), F32)

    xp = jnp.zeros((NP, 8), F32).at[:N, :2].set(x)
    W0 = jnp.zeros((8, HID), F32).at[:2].set(Ws[0])
    g0 = jnp.ones((1, 8), F32).at[0, :2].set(bn_g[0])
    bb0 = jnp.zeros((1, 8), F32).at[0, :2].set(bn_b[0])
    hlo, hhi = _dense([xp], W0, g0, bb0, bs[0].reshape(1, HID),
                      relu_in=False, out_widths=(32, 32))

    W11 = jnp.zeros((HID, HID), F32).at[:, :OUT].set(Ws[11])
    b11 = jnp.zeros((HID,), F32).at[:OUT].set(bs[11])
    Wst = jnp.stack(Ws[1:11] + [W11, Ws[1]])
    bst = jnp.stack(bs[1:11] + [b11, bs[1]]).reshape(12, 1, HID)
    gst = jnp.stack(bn_g[1:12] + [bn_g[1]]).reshape(12, 1, HID)
    nst = jnp.stack(bn_b[1:12] + [bn_b[1]]).reshape(12, 1, HID)

    def step(carry, ws):
        tlo, thi, _, _ = carry
        W, b, g, bb = ws
        rl, rh = _sc64(tlo, thi, rowp, col2, wp, z128)
        tlo2, thi2 = _dense([rl, rh], W, g, bb, b,
                            relu_in=True, out_widths=(32, 32))
        return (tlo2, thi2, rl, rh), None

    (_, _, rl, rh), _ = lax.scan(step, (hlo, hhi, hlo, hhi),
                                 (Wst, bst, gst, nst))
    hf = _hf_apply(rl, rh, fin_g.reshape(1, OUT), fin_b.reshape(1, OUT))

    fr, fc = _scgather(hf, rowp, colp)
    sums, xtx = _edge_stats(fr, fc)
    mu_e = sums.reshape(32) / E
    C = xtx / E - jnp.outer(mu_e, mu_e)
    W1r = jnp.roll(mW1, OUT, axis=0)

    def fold(W1):
        mu_z = mu_e @ W1 + mb1
        var_z = jnp.sum(W1 * (C @ W1), axis=0)
        s = mg / jnp.sqrt(var_z + EPS)
        t = (mb1 - mu_z) * s + mbt
        return W1 * s[None, :], t.reshape(1, MLP_HID)

    Ws1, t1 = fold(mW1)
    Ws2, t2 = fold(W1r)
    out = _edge_mlp(fr, fc, Ws1.astype(jnp.bfloat16), t1,
                    Ws2.astype(jnp.bfloat16), t2,
                    mW2.astype(jnp.bfloat16), mb2.reshape(1, 1))
    return out[:E]
